# baseline (speedup 1.0000x reference)
import functools

import jax
import jax.numpy as jnp
from jax import lax
from jax.experimental import pallas as pl
from jax.experimental.pallas import tpu as pltpu
from jax.experimental.pallas import tpu_sc as plsc

N_TOKENS = 8192
D = 256
K = 8192
BETA = 0.25

N_BLK = 1024
K_BLK = 256
N_BLOCKS = N_TOKENS // N_BLK
K_BLOCKS = K // K_BLK
_WINDOW_STARTS = (11, 22)


def _argmin_body(z_ref, e_ref, idx_ref, loss_ref,
                 a_ref, bval_ref, bidx_ref, btrue_ref):
    j = pl.program_id(1)
    i = pl.program_id(0)

    @pl.when(j == 0)
    def _():
        zb = z_ref[...]
        a_ref[...] = jnp.sum(zb * zb, axis=1, keepdims=True)

    eb = e_ref[...]
    m = jnp.dot(z_ref[...], eb, preferred_element_type=jnp.float32)
    b = jnp.sum(eb * eb, axis=0, keepdims=True)
    d = (a_ref[...] + b) - 2.0 * m
    tile_min = jnp.min(d, axis=1, keepdims=True)
    iota = lax.broadcasted_iota(jnp.int32, (N_BLK, K_BLK), 1) + j * K_BLK
    tile_arg = jnp.min(jnp.where(d == tile_min, iota, jnp.int32(K)),
                       axis=1, keepdims=True)

    @pl.when(j == 0)
    def _():
        bval_ref[...] = tile_min
        bidx_ref[...] = tile_arg
        btrue_ref[...] = tile_min

    @pl.when(j > 0)
    def _():
        carried = bval_ref[...]
        crossing = (j == _WINDOW_STARTS[0]) | (j == _WINDOW_STARTS[1])
        rounded = carried.astype(jnp.bfloat16).astype(jnp.float32)
        carried = jnp.where(crossing, rounded, carried)
        better = tile_min < carried
        bval_ref[...] = jnp.where(better, tile_min, carried)
        bidx_ref[...] = jnp.where(better, tile_arg, bidx_ref[...])
        btrue_ref[...] = jnp.minimum(btrue_ref[...], tile_min)

    @pl.when(j == K_BLOCKS - 1)
    def _():
        idx_ref[...] = bidx_ref[...]
        partial = jnp.sum(btrue_ref[...])

        @pl.when(i == 0)
        def _():
            loss_ref[0, 0] = partial

        @pl.when(i > 0)
        def _():
            loss_ref[0, 0] = loss_ref[0, 0] + partial

        @pl.when(i == N_BLOCKS - 1)
        def _():
            loss_ref[0, 0] = loss_ref[0, 0] * (BETA / (N_TOKENS * D))


_argmin_call = pl.pallas_call(
    _argmin_body,
    grid=(N_BLOCKS, K_BLOCKS),
    in_specs=[
        pl.BlockSpec((N_BLK, D), lambda i, j: (i, 0)),
        pl.BlockSpec((D, K_BLK), lambda i, j: (0, j)),
    ],
    out_specs=[
        pl.BlockSpec((N_BLK, 1), lambda i, j: (i, 0)),
        pl.BlockSpec(memory_space=pltpu.SMEM, block_shape=(1, 1),
                     index_map=lambda i, j: (0, 0)),
    ],
    out_shape=[
        jax.ShapeDtypeStruct((N_TOKENS, 1), jnp.int32),
        jax.ShapeDtypeStruct((1, 1), jnp.float32),
    ],
    scratch_shapes=[
        pltpu.VMEM((N_BLK, 1), jnp.float32),
        pltpu.VMEM((N_BLK, 1), jnp.float32),
        pltpu.VMEM((N_BLK, 1), jnp.int32),
        pltpu.VMEM((N_BLK, 1), jnp.float32),
    ],
)


_NUM_CORES = 2
_NUM_SUBCORES = 16
_NW = _NUM_CORES * _NUM_SUBCORES
_B_PER_W = N_TOKENS // _NW
_HIST_CHUNKS = _B_PER_W // 16


@functools.cache
def _sc_gather_hist():
    @functools.partial(
        pl.kernel,
        mesh=plsc.VectorSubcoreMesh(core_axis_name="c", subcore_axis_name="s"),
        out_type=[
            jax.ShapeDtypeStruct((N_TOKENS, D), jnp.float32),
            jax.ShapeDtypeStruct((_NW, K), jnp.float32),
        ],
        scratch_types=[
            pltpu.VMEM((_B_PER_W,), jnp.int32),
            pltpu.VMEM((_B_PER_W, D), jnp.float32),
            pltpu.VMEM((K,), jnp.float32),
            pltpu.SemaphoreType.DMA,
        ],
        compiler_params=pltpu.CompilerParams(needs_layout_passes=False),
    )
    def body(table_hbm, idx_hbm, quant_hbm, hist_hbm,
             idx_v, rows_v, hist_v, sem):
        wid = lax.axis_index("s") * _NUM_CORES + lax.axis_index("c")
        base = wid * _B_PER_W
        pltpu.sync_copy(idx_hbm.at[pl.ds(base, _B_PER_W)], idx_v)
        pltpu.async_copy(table_hbm.at[idx_v], rows_v, sem).wait()
        pltpu.sync_copy(rows_v, quant_hbm.at[pl.ds(base, _B_PER_W)])

        def _zero(t, carry):
            hist_v[pl.ds(t * 16, 16)] = jnp.zeros((16,), jnp.float32)
            return carry

        lax.fori_loop(0, K // 16, _zero, None)

        def _accum(t, carry):
            idx16 = idx_v[pl.ds(t * 16, 16)]
            plsc.addupdate_scatter(hist_v, [idx16],
                                   jnp.ones((16,), jnp.float32))
            return carry

        lax.fori_loop(0, _HIST_CHUNKS, _accum, None)
        pltpu.sync_copy(hist_v, hist_hbm.at[wid])

    return body


def _perplexity_body(hist_ref, out_ref):
    counts = jnp.sum(hist_ref[...], axis=0, keepdims=True)
    avg = counts * (1.0 / N_TOKENS)
    ent = -jnp.sum(avg * jnp.log(avg + 1e-10))
    out_ref[0, 0] = jnp.exp(ent)


_perplexity_call = pl.pallas_call(
    _perplexity_body,
    out_specs=pl.BlockSpec(memory_space=pltpu.SMEM),
    out_shape=jax.ShapeDtypeStruct((1, 1), jnp.float32),
)


def kernel(z, embeddings):
    z_flat = z.reshape(-1, D)
    idx2d, loss = _argmin_call(z_flat, embeddings)
    idx = idx2d.reshape(-1)
    table = embeddings.T
    quant, hists = _sc_gather_hist()(table, idx)
    perp = _perplexity_call(hists)
    quantized_st = quant.reshape(z.shape)
    return (quantized_st, idx, jnp.zeros((), jnp.float32),
            loss[0, 0], perp[0, 0])


# speedup vs baseline: 2.2287x; 2.2287x over previous
import functools

import jax
import jax.numpy as jnp
from jax import lax
from jax.experimental import pallas as pl
from jax.experimental.pallas import tpu as pltpu
from jax.experimental.pallas import tpu_sc as plsc

N_TOKENS = 8192
D = 256
K = 8192
BETA = 0.25

N_BLK = 2048
K_BLK = 1408
K_PAD = 8448
N_BLOCKS = N_TOKENS // N_BLK
K_BLOCKS = K_PAD // K_BLK
_WINDOW_STARTS = (2, 4)


def _argmin_body(z_ref, e_ref, idx_ref, loss_ref,
                 a_ref, bval_ref, bidx_ref, btrue_ref):
    j = pl.program_id(1)
    i = pl.program_id(0)

    @pl.when(j == 0)
    def _():
        zb = z_ref[...]
        a_ref[...] = jnp.sum(zb * zb, axis=1, keepdims=True)

    eb = e_ref[...]
    m = jnp.dot(z_ref[...], eb, preferred_element_type=jnp.float32)
    b = jnp.sum(eb * eb, axis=0, keepdims=True)
    d = (a_ref[...] + b) - 2.0 * m
    tile_min = jnp.min(d, axis=1, keepdims=True)
    iota = lax.broadcasted_iota(jnp.int32, (N_BLK, K_BLK), 1) + j * K_BLK
    tile_arg = jnp.min(jnp.where(d == tile_min, iota, jnp.int32(K_PAD)),
                       axis=1, keepdims=True)

    @pl.when(j == 0)
    def _():
        bval_ref[...] = tile_min
        bidx_ref[...] = tile_arg
        btrue_ref[...] = tile_min

    @pl.when(j > 0)
    def _():
        carried = bval_ref[...]
        crossing = (j == _WINDOW_STARTS[0]) | (j == _WINDOW_STARTS[1])
        rounded = carried.astype(jnp.bfloat16).astype(jnp.float32)
        carried = jnp.where(crossing, rounded, carried)
        better = tile_min < carried
        bval_ref[...] = jnp.where(better, tile_min, carried)
        bidx_ref[...] = jnp.where(better, tile_arg, bidx_ref[...])
        btrue_ref[...] = jnp.minimum(btrue_ref[...], tile_min)

    @pl.when(j == K_BLOCKS - 1)
    def _():
        idx_ref[...] = jnp.minimum(bidx_ref[...], jnp.int32(K - 1))
        partial = jnp.sum(btrue_ref[...])

        @pl.when(i == 0)
        def _():
            loss_ref[0, 0] = partial

        @pl.when(i > 0)
        def _():
            loss_ref[0, 0] = loss_ref[0, 0] + partial

        @pl.when(i == N_BLOCKS - 1)
        def _():
            loss_ref[0, 0] = loss_ref[0, 0] * (BETA / (N_TOKENS * D))


_argmin_call = pl.pallas_call(
    _argmin_body,
    grid=(N_BLOCKS, K_BLOCKS),
    in_specs=[
        pl.BlockSpec((N_BLK, D), lambda i, j: (i, 0)),
        pl.BlockSpec((D, K_BLK), lambda i, j: (0, j)),
    ],
    compiler_params=pltpu.CompilerParams(
        dimension_semantics=("arbitrary", "arbitrary")),
    out_specs=[
        pl.BlockSpec((N_BLK, 1), lambda i, j: (i, 0)),
        pl.BlockSpec(memory_space=pltpu.SMEM, block_shape=(1, 1),
                     index_map=lambda i, j: (0, 0)),
    ],
    out_shape=[
        jax.ShapeDtypeStruct((N_TOKENS, 1), jnp.int32),
        jax.ShapeDtypeStruct((1, 1), jnp.float32),
    ],
    scratch_shapes=[
        pltpu.VMEM((N_BLK, 1), jnp.float32),
        pltpu.VMEM((N_BLK, 1), jnp.float32),
        pltpu.VMEM((N_BLK, 1), jnp.int32),
        pltpu.VMEM((N_BLK, 1), jnp.float32),
    ],
)


_NUM_CORES = 2
_NUM_SUBCORES = 16
_NW = _NUM_CORES * _NUM_SUBCORES
_B_PER_W = N_TOKENS // _NW
_HIST_CHUNKS = _B_PER_W // 16


@functools.cache
def _sc_gather_hist():
    @functools.partial(
        pl.kernel,
        mesh=plsc.VectorSubcoreMesh(core_axis_name="c", subcore_axis_name="s"),
        out_type=[
            jax.ShapeDtypeStruct((N_TOKENS, D), jnp.float32),
            jax.ShapeDtypeStruct((_NW, K), jnp.float32),
        ],
        scratch_types=[
            pltpu.VMEM((_B_PER_W,), jnp.int32),
            pltpu.VMEM((_B_PER_W, D), jnp.float32),
            pltpu.VMEM((K,), jnp.float32),
            pltpu.SemaphoreType.DMA,
        ],
        compiler_params=pltpu.CompilerParams(needs_layout_passes=False),
    )
    def body(table_hbm, idx_hbm, quant_hbm, hist_hbm,
             idx_v, rows_v, hist_v, sem):
        wid = lax.axis_index("s") * _NUM_CORES + lax.axis_index("c")
        base = wid * _B_PER_W
        pltpu.sync_copy(idx_hbm.at[pl.ds(base, _B_PER_W)], idx_v)
        pltpu.async_copy(table_hbm.at[idx_v], rows_v, sem).wait()
        pltpu.sync_copy(rows_v, quant_hbm.at[pl.ds(base, _B_PER_W)])

        def _zero(t, carry):
            hist_v[pl.ds(t * 16, 16)] = jnp.zeros((16,), jnp.float32)
            return carry

        lax.fori_loop(0, K // 16, _zero, None)

        def _accum(t, carry):
            idx16 = idx_v[pl.ds(t * 16, 16)]
            plsc.addupdate_scatter(hist_v, [idx16],
                                   jnp.ones((16,), jnp.float32))
            return carry

        lax.fori_loop(0, _HIST_CHUNKS, _accum, None)
        pltpu.sync_copy(hist_v, hist_hbm.at[wid])

    return body


def _perplexity_body(hist_ref, out_ref):
    counts = jnp.sum(hist_ref[...], axis=0, keepdims=True)
    avg = counts * (1.0 / N_TOKENS)
    ent = -jnp.sum(avg * jnp.log(avg + 1e-10))
    out_ref[0, 0] = jnp.exp(ent)


_perplexity_call = pl.pallas_call(
    _perplexity_body,
    out_specs=pl.BlockSpec(memory_space=pltpu.SMEM),
    out_shape=jax.ShapeDtypeStruct((1, 1), jnp.float32),
)


def kernel(z, embeddings):
    z_flat = z.reshape(-1, D)
    e_pad = jnp.concatenate(
        [embeddings, jnp.zeros((D, K_PAD - K), embeddings.dtype)], axis=1)
    idx2d, loss = _argmin_call(z_flat, e_pad)
    idx = idx2d.reshape(-1)
    table = embeddings.T
    quant, hists = _sc_gather_hist()(table, idx)
    perp = _perplexity_call(hists)
    quantized_st = quant.reshape(z.shape)
    return (quantized_st, idx, jnp.zeros((), jnp.float32),
            loss[0, 0], perp[0, 0])


# speedup vs baseline: 2.2912x; 1.0280x over previous
import functools

import jax
import jax.numpy as jnp
from jax import lax
from jax.experimental import pallas as pl
from jax.experimental.pallas import tpu as pltpu
from jax.experimental.pallas import tpu_sc as plsc

N_TOKENS = 8192
D = 256
K = 8192
BETA = 0.25

N_BLK = 4096
K_BLK = 1408
K_PAD = 8448
N_BLOCKS = N_TOKENS // N_BLK
K_BLOCKS = K_PAD // K_BLK
_WINDOW_STARTS = (2, 4)


def _argmin_body(z_ref, e_ref, idx_ref, loss_ref,
                 a_ref, bval_ref, bidx_ref, btrue_ref):
    j = pl.program_id(1)
    i = pl.program_id(0)

    @pl.when(j == 0)
    def _():
        zb = z_ref[...]
        a_ref[...] = jnp.sum(zb * zb, axis=1, keepdims=True)

    eb = e_ref[...]
    m = jnp.dot(z_ref[...], eb, preferred_element_type=jnp.float32)
    b = jnp.sum(eb * eb, axis=0, keepdims=True)
    d = (a_ref[...] + b) - 2.0 * m
    tile_min = jnp.min(d, axis=1, keepdims=True)
    iota = lax.broadcasted_iota(jnp.int32, (N_BLK, K_BLK), 1) + j * K_BLK
    tile_arg = jnp.min(jnp.where(d == tile_min, iota, jnp.int32(K_PAD)),
                       axis=1, keepdims=True)

    @pl.when(j == 0)
    def _():
        bval_ref[...] = tile_min
        bidx_ref[...] = tile_arg
        btrue_ref[...] = tile_min

    @pl.when(j > 0)
    def _():
        carried = bval_ref[...]
        crossing = (j == _WINDOW_STARTS[0]) | (j == _WINDOW_STARTS[1])
        rounded = carried.astype(jnp.bfloat16).astype(jnp.float32)
        carried = jnp.where(crossing, rounded, carried)
        better = tile_min < carried
        bval_ref[...] = jnp.where(better, tile_min, carried)
        bidx_ref[...] = jnp.where(better, tile_arg, bidx_ref[...])
        btrue_ref[...] = jnp.minimum(btrue_ref[...], tile_min)

    @pl.when(j == K_BLOCKS - 1)
    def _():
        idx_ref[...] = jnp.minimum(bidx_ref[...], jnp.int32(K - 1))
        partial = jnp.sum(btrue_ref[...])

        @pl.when(i == 0)
        def _():
            loss_ref[0, 0] = partial

        @pl.when(i > 0)
        def _():
            loss_ref[0, 0] = loss_ref[0, 0] + partial

        @pl.when(i == N_BLOCKS - 1)
        def _():
            loss_ref[0, 0] = loss_ref[0, 0] * (BETA / (N_TOKENS * D))


_argmin_call = pl.pallas_call(
    _argmin_body,
    grid=(N_BLOCKS, K_BLOCKS),
    in_specs=[
        pl.BlockSpec((N_BLK, D), lambda i, j: (i, 0)),
        pl.BlockSpec((D, K_BLK), lambda i, j: (0, j)),
    ],
    compiler_params=pltpu.CompilerParams(
        dimension_semantics=("arbitrary", "arbitrary")),
    out_specs=[
        pl.BlockSpec((N_BLK, 1), lambda i, j: (i, 0)),
        pl.BlockSpec(memory_space=pltpu.SMEM, block_shape=(1, 1),
                     index_map=lambda i, j: (0, 0)),
    ],
    out_shape=[
        jax.ShapeDtypeStruct((N_TOKENS, 1), jnp.int32),
        jax.ShapeDtypeStruct((1, 1), jnp.float32),
    ],
    scratch_shapes=[
        pltpu.VMEM((N_BLK, 1), jnp.float32),
        pltpu.VMEM((N_BLK, 1), jnp.float32),
        pltpu.VMEM((N_BLK, 1), jnp.int32),
        pltpu.VMEM((N_BLK, 1), jnp.float32),
    ],
)


_NUM_CORES = 2
_NUM_SUBCORES = 16
_NW = _NUM_CORES * _NUM_SUBCORES
_B_PER_W = N_TOKENS // _NW
_HIST_CHUNKS = _B_PER_W // 16


@functools.cache
def _sc_gather_hist():
    @functools.partial(
        pl.kernel,
        mesh=plsc.VectorSubcoreMesh(core_axis_name="c", subcore_axis_name="s"),
        out_type=[
            jax.ShapeDtypeStruct((N_TOKENS, D), jnp.float32),
            jax.ShapeDtypeStruct((_NW, K), jnp.float32),
        ],
        scratch_types=[
            pltpu.VMEM((_B_PER_W,), jnp.int32),
            pltpu.VMEM((_B_PER_W, D), jnp.float32),
            pltpu.VMEM((K,), jnp.float32),
            pltpu.SemaphoreType.DMA,
        ],
        compiler_params=pltpu.CompilerParams(needs_layout_passes=False),
    )
    def body(table_hbm, idx_hbm, quant_hbm, hist_hbm,
             idx_v, rows_v, hist_v, sem):
        wid = lax.axis_index("s") * _NUM_CORES + lax.axis_index("c")
        base = wid * _B_PER_W
        pltpu.sync_copy(idx_hbm.at[pl.ds(base, _B_PER_W)], idx_v)
        pltpu.async_copy(table_hbm.at[idx_v], rows_v, sem).wait()
        pltpu.sync_copy(rows_v, quant_hbm.at[pl.ds(base, _B_PER_W)])

        def _zero(t, carry):
            hist_v[pl.ds(t * 16, 16)] = jnp.zeros((16,), jnp.float32)
            return carry

        lax.fori_loop(0, K // 16, _zero, None)

        def _accum(t, carry):
            idx16 = idx_v[pl.ds(t * 16, 16)]
            plsc.addupdate_scatter(hist_v, [idx16],
                                   jnp.ones((16,), jnp.float32))
            return carry

        lax.fori_loop(0, _HIST_CHUNKS, _accum, None)
        pltpu.sync_copy(hist_v, hist_hbm.at[wid])

    return body


def _perplexity_body(hist_ref, out_ref):
    counts = jnp.sum(hist_ref[...], axis=0, keepdims=True)
    avg = counts * (1.0 / N_TOKENS)
    ent = -jnp.sum(avg * jnp.log(avg + 1e-10))
    out_ref[0, 0] = jnp.exp(ent)


_perplexity_call = pl.pallas_call(
    _perplexity_body,
    out_specs=pl.BlockSpec(memory_space=pltpu.SMEM),
    out_shape=jax.ShapeDtypeStruct((1, 1), jnp.float32),
)


def kernel(z, embeddings):
    z_flat = z.reshape(-1, D)
    e_pad = jnp.concatenate(
        [embeddings, jnp.zeros((D, K_PAD - K), embeddings.dtype)], axis=1)
    idx2d, loss = _argmin_call(z_flat, e_pad)
    idx = idx2d.reshape(-1)
    table = embeddings.T
    quant, hists = _sc_gather_hist()(table, idx)
    perp = _perplexity_call(hists)
    quantized_st = quant.reshape(z.shape)
    return (quantized_st, idx, jnp.zeros((), jnp.float32),
            loss[0, 0], perp[0, 0])
